# Initial kernel scaffold; baseline (speedup 1.0000x reference)
import jax
import jax.numpy as jnp
from jax.experimental import pallas as pl


def kernel(x, psm, mask):
    raise NotImplementedError("write your pallas kernel here")


# speedup vs baseline: 2.9315x; 2.9315x over previous
import jax
import jax.numpy as jnp
from jax.experimental import pallas as pl
from jax.experimental.pallas import tpu as pltpu

K = 1024


def _mask_body(mask_ref, cav_ref, ego_ref, out_ref):
    l = pl.program_id(1)
    b = pl.program_id(0)
    cav = cav_ref[0, 0]
    ego = ego_ref[0, 0]
    r = jnp.maximum(cav[0] - ego[0], cav[1] - ego[1])
    bits = jax.lax.bitcast_convert_type(r, jnp.uint32)
    neg = bits >= jnp.uint32(0x80000000)
    key = jnp.where(neg, ~bits, bits | jnp.uint32(0x80000000))

    def step(i, prefix):
        cand = prefix | (jnp.uint32(1) << (jnp.uint32(31) - i.astype(jnp.uint32)))
        cnt = jnp.sum((key >= cand).astype(jnp.int32))
        return jnp.where(cnt >= K, cand, prefix)

    thr = jax.lax.fori_loop(0, 32, step, jnp.uint32(0))
    m = (mask_ref[b, l] != 0).astype(jnp.float32)
    fm = (key >= thr).astype(jnp.float32) * m
    out_ref[0, 0] = jnp.where(l == 0, jnp.float32(1.0), fm)


def _apply_body(x_ref, m_ref, o_ref):
    m = m_ref[...]
    o_ref[...] = x_ref[...] * m[:, None, :, :]


def kernel(x, psm, mask):
    B, L, C, H, W = x.shape
    A = psm.shape[2]
    del A

    maskplane = pl.pallas_call(
        _mask_body,
        grid=(B, L),
        in_specs=[
            pl.BlockSpec(memory_space=pltpu.SMEM),
            pl.BlockSpec((1, 1, 2, H, W), lambda b, l: (b, l, 0, 0, 0)),
            pl.BlockSpec((1, 1, 2, H, W), lambda b, l: (b, 0, 0, 0, 0)),
        ],
        out_specs=pl.BlockSpec((1, 1, H, W), lambda b, l: (b, l, 0, 0)),
        out_shape=jax.ShapeDtypeStruct((B, L, H, W), jnp.float32),
    )(mask, psm, psm)

    CB = 16
    xf = x.reshape(B * L, C, H, W)
    mp = maskplane.reshape(B * L, H, W)
    out = pl.pallas_call(
        _apply_body,
        grid=(B * L, C // CB),
        in_specs=[
            pl.BlockSpec((1, CB, H, W), lambda p, c: (p, c, 0, 0)),
            pl.BlockSpec((1, H, W), lambda p, c: (p, 0, 0)),
        ],
        out_specs=pl.BlockSpec((1, CB, H, W), lambda p, c: (p, c, 0, 0)),
        out_shape=jax.ShapeDtypeStruct((B * L, C, H, W), jnp.float32),
    )(xf, mp)
    return out.reshape(B, L, C, H, W)
